# baseline (speedup 1.0000x reference)
import functools

import jax
import jax.numpy as jnp
from jax import lax
from jax.experimental import pallas as pl
from jax.experimental.pallas import tpu as pltpu
from jax.experimental.pallas import tpu_sc as plsc

VOCAB = 1000000
EMBED_DIM = 32
BATCH = 16384

ROWS_PER_SLICE = 128 // EMBED_DIM
TABLE_MAJOR = VOCAB // ROWS_PER_SLICE

NC = 2
NS = 16
L = 16
NW = NC * NS
B_PER_W = BATCH // NW
CHUNK = 128
N_CHUNKS = B_PER_W // CHUNK
GROUPS = CHUNK // L


def _sc_body(emb_hbm, tgt_hbm, ctx_hbm, out_hbm,
             idx_t, idx_c, sup_t, sup_c, rows_t, rows_c, out_v, sem):
    wid = lax.axis_index("s") * NC + lax.axis_index("c")
    base = wid * B_PER_W

    pltpu.sync_copy(tgt_hbm.at[pl.ds(base, B_PER_W)], idx_t)
    pltpu.sync_copy(ctx_hbm.at[pl.ds(base, B_PER_W)], idx_c)

    for g in range(B_PER_W // L):
        sl = pl.ds(g * L, L)
        sup_t[sl] = lax.shift_right_logical(idx_t[sl], ROWS_PER_SLICE // 2)
        sup_c[sl] = lax.shift_right_logical(idx_c[sl], ROWS_PER_SLICE // 2)

    def start(j, buf):
        sl = pl.ds(j * CHUNK, CHUNK)
        return (pltpu.async_copy(emb_hbm.at[sup_t.at[sl]], rows_t.at[buf], sem),
                pltpu.async_copy(emb_hbm.at[sup_c.at[sl]], rows_c.at[buf], sem))

    lanes = jnp.arange(L, dtype=jnp.int32)
    pending = start(0, 0)
    for j in range(N_CHUNKS):
        nxt = start(j + 1, (j + 1) % 2) if j + 1 < N_CHUNKS else None
        for c in pending:
            c.wait()
        buf = j % 2
        for g in range(GROUPS):
            pos = g * L + lanes
            off_t = (idx_t[pl.ds(j * CHUNK + g * L, L)] & (ROWS_PER_SLICE - 1)) * EMBED_DIM
            off_c = (idx_c[pl.ds(j * CHUNK + g * L, L)] & (ROWS_PER_SLICE - 1)) * EMBED_DIM
            acc = jnp.zeros((L,), jnp.float32)
            for col in range(EMBED_DIM):
                vt = plsc.load_gather(rows_t.at[buf], [pos, off_t + col])
                vc = plsc.load_gather(rows_c.at[buf], [pos, off_c + col])
                acc = acc + vt * vc
            out_v[pl.ds(j * CHUNK + g * L, L)] = acc
        pending = nxt

    pltpu.sync_copy(out_v, out_hbm.at[pl.ds(base, B_PER_W)])


@jax.jit
def _word2vec_score(target_word, context_word, embeddings):
    emb_view = embeddings.reshape(TABLE_MAJOR, 128)
    mesh = plsc.VectorSubcoreMesh(core_axis_name="c", subcore_axis_name="s")
    k = functools.partial(
        pl.kernel,
        mesh=mesh,
        compiler_params=pltpu.CompilerParams(needs_layout_passes=False),
        out_type=jax.ShapeDtypeStruct((BATCH,), jnp.float32),
        scratch_types=[
            pltpu.VMEM((B_PER_W,), jnp.int32),
            pltpu.VMEM((B_PER_W,), jnp.int32),
            pltpu.VMEM((B_PER_W,), jnp.int32),
            pltpu.VMEM((B_PER_W,), jnp.int32),
            pltpu.VMEM((2, CHUNK, 128), jnp.float32),
            pltpu.VMEM((2, CHUNK, 128), jnp.float32),
            pltpu.VMEM((B_PER_W,), jnp.float32),
            pltpu.SemaphoreType.DMA,
        ],
    )(_sc_body)
    return k(emb_view, target_word, context_word)


def kernel(target_word, context_word, embeddings):
    return _word2vec_score(target_word.astype(jnp.int32),
                           context_word.astype(jnp.int32),
                           embeddings)


# speedup vs baseline: 1.6071x; 1.6071x over previous
import functools

import jax
import jax.numpy as jnp
from jax import lax
from jax.experimental import pallas as pl
from jax.experimental.pallas import tpu as pltpu
from jax.experimental.pallas import tpu_sc as plsc

VOCAB = 1000000
EMBED_DIM = 32
BATCH = 16384

NC = 2
NS = 16
L = 16
NW = NC * NS
B_PER_W = BATCH // NW
CHUNK = 128
N_CHUNKS = B_PER_W // CHUNK
GROUPS = CHUNK // L


def _sc_body(emb_hbm, tgt_hbm, ctx_hbm, out_hbm,
             idx_tv, idx_cv, rows_t, rows_c, out_v,
             sem_t, sem_c):
    wid = lax.axis_index("s") * NC + lax.axis_index("c")
    base = wid * B_PER_W

    pltpu.sync_copy(tgt_hbm.at[pl.ds(base, B_PER_W)], idx_tv)
    pltpu.sync_copy(ctx_hbm.at[pl.ds(base, B_PER_W)], idx_cv)

    def fire(j, buf):
        def issue(g, _):
            vt = idx_tv[pl.ds(j * CHUNK + g * L, L)]
            vc = idx_cv[pl.ds(j * CHUNK + g * L, L)]
            for r in range(L):
                pltpu.async_copy(emb_hbm.at[pl.ds(vt[r], 1)],
                                 rows_t.at[buf, pl.ds(g * L + r, 1)], sem_t)
                pltpu.async_copy(emb_hbm.at[pl.ds(vc[r], 1)],
                                 rows_c.at[buf, pl.ds(g * L + r, 1)], sem_c)
            return 0
        lax.fori_loop(0, GROUPS, issue, 0)

    def drain(buf):
        pltpu.make_async_copy(
            emb_hbm.at[pl.ds(0, CHUNK)], rows_t.at[buf], sem_t).wait()
        pltpu.make_async_copy(
            emb_hbm.at[pl.ds(0, CHUNK)], rows_c.at[buf], sem_c).wait()

    lanes = jnp.arange(L, dtype=jnp.int32)
    fire(0, 0)
    for j in range(N_CHUNKS):
        if j + 1 < N_CHUNKS:
            fire(j + 1, (j + 1) % 2)
        buf = j % 2
        drain(buf)
        for g in range(GROUPS):
            pos = g * L + lanes
            acc = jnp.zeros((L,), jnp.float32)
            for col in range(EMBED_DIM):
                colv = jnp.full((L,), col, jnp.int32)
                vt = plsc.load_gather(rows_t.at[buf], [pos, colv])
                vc = plsc.load_gather(rows_c.at[buf], [pos, colv])
                acc = acc + vt * vc
            out_v[pl.ds(j * CHUNK + g * L, L)] = acc

    pltpu.sync_copy(out_v, out_hbm.at[pl.ds(base, B_PER_W)])


@jax.jit
def _word2vec_score(target_word, context_word, embeddings):
    mesh = plsc.VectorSubcoreMesh(core_axis_name="c", subcore_axis_name="s")
    k = functools.partial(
        pl.kernel,
        mesh=mesh,
        compiler_params=pltpu.CompilerParams(needs_layout_passes=False),
        out_type=jax.ShapeDtypeStruct((BATCH,), jnp.float32),
        scratch_types=[
            pltpu.VMEM((B_PER_W,), jnp.int32),
            pltpu.VMEM((B_PER_W,), jnp.int32),
            pltpu.VMEM((2, CHUNK, EMBED_DIM), jnp.float32),
            pltpu.VMEM((2, CHUNK, EMBED_DIM), jnp.float32),
            pltpu.VMEM((B_PER_W,), jnp.float32),
            pltpu.SemaphoreType.DMA,
            pltpu.SemaphoreType.DMA,
        ],
    )(_sc_body)
    return k(embeddings, target_word, context_word)


def kernel(target_word, context_word, embeddings):
    return _word2vec_score(target_word.astype(jnp.int32),
                           context_word.astype(jnp.int32),
                           embeddings)
